# Initial kernel scaffold; baseline (speedup 1.0000x reference)
import jax
import jax.numpy as jnp
from jax.experimental import pallas as pl


def kernel(x, edge_index, W1_l, b1_l, W1_r, W2_l, b2_l, W2_r):
    raise NotImplementedError("write your pallas kernel here")


# speedup vs baseline: 6.6121x; 6.6121x over previous
import functools

import jax
import jax.numpy as jnp
from jax import lax
from jax.experimental import pallas as pl
from jax.experimental.pallas import tpu as pltpu
from jax.experimental.pallas import tpu_sc as plsc

NODES = 10000
PADN = 10240
EDGES = 320000
FEAT = 128
CHUNK = 128
NSC = 2
NSUB = 16
ROWS_PER_SUB = PADN // NSUB
N_CHUNKS = EDGES // CHUNK
PER_CORE = N_CHUNKS // NSC
ITERS = -(-PER_CORE // NSUB)


CROWS = PADN // FEAT


@functools.cache
def _make_sc_agg(with_counts):
  mesh = plsc.VectorSubcoreMesh(
      core_axis_name="c", subcore_axis_name="s", num_cores=NSC,
      num_subcores=NSUB)

  out_type = [jax.ShapeDtypeStruct((NSC, PADN, FEAT), jnp.float32)]
  scratch = [
      pltpu.VMEM((CHUNK,), jnp.int32),
      pltpu.VMEM((CHUNK,), jnp.int32),
      pltpu.VMEM((CHUNK, FEAT), jnp.float32),
      pltpu.VMEM((128, FEAT), jnp.float32),
      pltpu.VMEM_SHARED((PADN, FEAT), jnp.float32),
      pltpu.SemaphoreType.DMA,
  ]
  if with_counts:
    out_type.append(jax.ShapeDtypeStruct((NSC, CROWS, FEAT), jnp.int32))
    scratch += [
        pltpu.VMEM((CROWS, FEAT), jnp.int32),
        pltpu.VMEM((CROWS,), jnp.int32),
        pltpu.VMEM_SHARED((CROWS, FEAT), jnp.int32),
    ]

  @functools.partial(
      pl.kernel, out_type=out_type, mesh=mesh, scratch_types=scratch,
      compiler_params=pltpu.CompilerParams(needs_layout_passes=False))
  def sc_agg(table, srci, dsti, *refs):
    if with_counts:
      part, cntp, idx_s, idx_d, rows, zbuf, acc, sem, cloc, iota_r, cacc = refs
    else:
      part, idx_s, idx_d, rows, zbuf, acc, sem = refs
    c = lax.axis_index("c")
    s = lax.axis_index("s")

    zv = jnp.zeros((16,), jnp.float32)

    def zrow(i, carry):
      def zcol(j, carry2):
        zbuf[i, pl.ds(j * 16, 16)] = zv
        return carry2
      return lax.fori_loop(0, FEAT // 16, zcol, carry)

    lax.fori_loop(0, 128, zrow, 0)
    for r in range(ROWS_PER_SUB // 128):
      pltpu.sync_copy(zbuf, acc.at[pl.ds(s * ROWS_PER_SUB + r * 128, 128)])

    if with_counts:
      zi = jnp.zeros((16,), jnp.int32)

      def czrow(i, carry):
        def czcol(j, carry2):
          cloc[i, pl.ds(j * 16, 16)] = zi
          return carry2
        return lax.fori_loop(0, FEAT // 16, czcol, carry)

      lax.fori_loop(0, CROWS, czrow, 0)
      for k in range(CROWS // 16):
        iota_r[pl.ds(k * 16, 16)] = lax.iota(jnp.int32, 16) + k * 16
      @pl.when(s < CROWS // 8)
      def _():
        pltpu.sync_copy(cloc.at[pl.ds(0, 8)], cacc.at[pl.ds(s * 8, 8)])
    plsc.subcore_barrier()

    def body(k, carry):
      rel = k * NSUB + s

      @pl.when(rel < PER_CORE)
      def _():
        base = (c * PER_CORE + rel) * CHUNK
        pltpu.sync_copy(srci.at[pl.ds(base, CHUNK)], idx_s)
        pltpu.sync_copy(dsti.at[pl.ds(base, CHUNK)], idx_d)
        pltpu.async_copy(table.at[idx_s], rows, sem).wait()
        pltpu.sync_copy(rows, acc.at[idx_d], add=True)
        if with_counts:
          for i in range(CHUNK // 16):
            v = idx_d[pl.ds(i * 16, 16)]
            row = lax.shift_right_logical(v, 7)
            col = jnp.bitwise_and(v, 127)
            cv, last = plsc.scan_count(v)
            plsc.addupdate_scatter(cloc, [row, col], cv, mask=last)

      return carry

    lax.fori_loop(0, ITERS, body, 0)

    if with_counts:
      pltpu.sync_copy(cloc, cacc.at[iota_r], add=True)
    plsc.subcore_barrier()

    pltpu.sync_copy(
        acc.at[pl.ds(s * ROWS_PER_SUB, ROWS_PER_SUB)],
        part.at[c, pl.ds(s * ROWS_PER_SUB, ROWS_PER_SUB)])
    if with_counts:
      @pl.when(s < CROWS // 8)
      def _():
        pltpu.sync_copy(cacc.at[pl.ds(s * 8, 8)],
                        cntp.at[c, pl.ds(s * 8, 8)])

  return sc_agg


_BLK = 1024
_GRID = (-(-NODES // _BLK),)


def _tc1_body(x_ref, wl_ref, wr_ref, b_ref, xl_ref, xr_ref):
  xb = x_ref[...]
  xl_ref[...] = jnp.dot(xb, wl_ref[...], preferred_element_type=jnp.float32)
  xr_ref[...] = (
      jnp.dot(xb, wr_ref[...], preferred_element_type=jnp.float32)
      + b_ref[...])


_tc1 = pl.pallas_call(
    _tc1_body,
    grid=_GRID,
    in_specs=[
        pl.BlockSpec((_BLK, FEAT), lambda i: (i, 0)),
        pl.BlockSpec((FEAT, FEAT), lambda i: (0, 0)),
        pl.BlockSpec((FEAT, FEAT), lambda i: (0, 0)),
        pl.BlockSpec((1, FEAT), lambda i: (0, 0)),
    ],
    out_specs=[
        pl.BlockSpec((_BLK, FEAT), lambda i: (i, 0)),
        pl.BlockSpec((_BLK, FEAT), lambda i: (i, 0)),
    ],
    out_shape=[
        jax.ShapeDtypeStruct((NODES, FEAT), jnp.float32),
        jax.ShapeDtypeStruct((NODES, FEAT), jnp.float32),
    ],
)


def _tc2_body(p0_ref, p1_ref, cnt_ref, xr_ref, wl_ref, wr_ref, b_ref,
              xl2_ref, xr2_ref):
  inv = 1.0 / jnp.maximum(cnt_ref[...], 1.0)
  x1 = (p0_ref[...] + p1_ref[...]) * inv + xr_ref[...]
  xl2_ref[...] = jnp.dot(x1, wl_ref[...], preferred_element_type=jnp.float32)
  xr2_ref[...] = (
      jnp.dot(x1, wr_ref[...], preferred_element_type=jnp.float32)
      + b_ref[...])


_tc2 = pl.pallas_call(
    _tc2_body,
    grid=_GRID,
    in_specs=[
        pl.BlockSpec((_BLK, FEAT), lambda i: (i, 0)),
        pl.BlockSpec((_BLK, FEAT), lambda i: (i, 0)),
        pl.BlockSpec((_BLK, FEAT), lambda i: (i, 0)),
        pl.BlockSpec((_BLK, FEAT), lambda i: (i, 0)),
        pl.BlockSpec((FEAT, FEAT), lambda i: (0, 0)),
        pl.BlockSpec((FEAT, FEAT), lambda i: (0, 0)),
        pl.BlockSpec((1, FEAT), lambda i: (0, 0)),
    ],
    out_specs=[
        pl.BlockSpec((_BLK, FEAT), lambda i: (i, 0)),
        pl.BlockSpec((_BLK, FEAT), lambda i: (i, 0)),
    ],
    out_shape=[
        jax.ShapeDtypeStruct((NODES, FEAT), jnp.float32),
        jax.ShapeDtypeStruct((NODES, FEAT), jnp.float32),
    ],
)


def _tc3_body(q0_ref, q1_ref, cnt_ref, xr_ref, out_ref):
  inv = 1.0 / jnp.maximum(cnt_ref[...], 1.0)
  out_ref[...] = (q0_ref[...] + q1_ref[...]) * inv + xr_ref[...]


_tc3 = pl.pallas_call(
    _tc3_body,
    grid=_GRID,
    in_specs=[
        pl.BlockSpec((_BLK, FEAT), lambda i: (i, 0)),
        pl.BlockSpec((_BLK, FEAT), lambda i: (i, 0)),
        pl.BlockSpec((_BLK, FEAT), lambda i: (i, 0)),
        pl.BlockSpec((_BLK, FEAT), lambda i: (i, 0)),
    ],
    out_specs=pl.BlockSpec((_BLK, FEAT), lambda i: (i, 0)),
    out_shape=jax.ShapeDtypeStruct((NODES, FEAT), jnp.float32),
)


def kernel(x, edge_index, W1_l, b1_l, W1_r, W2_l, b2_l, W2_r):
  src = edge_index[0]
  dst = edge_index[1]

  xl1, xr1b = _tc1(x, W1_l, W1_r, b1_l.reshape(1, FEAT))

  part1, cntp = _make_sc_agg(True)(xl1, src, dst)

  p0 = part1[0, :NODES]
  p1 = part1[1, :NODES]
  cnt = (cntp[0] + cntp[1]).astype(jnp.float32).reshape(PADN)[:NODES]
  cnt_b = jnp.broadcast_to(cnt[:, None], (NODES, FEAT))

  xl2, xr2b = _tc2(p0, p1, cnt_b, xr1b, W2_l, W2_r, b2_l.reshape(1, FEAT))

  (part2,) = _make_sc_agg(False)(xl2, src, dst)

  return _tc3(part2[0, :NODES], part2[1, :NODES], cnt_b, xr2b)


# speedup vs baseline: 8.8813x; 1.3432x over previous
import functools

import jax
import jax.numpy as jnp
from jax import lax
from jax.experimental import pallas as pl
from jax.experimental.pallas import tpu as pltpu
from jax.experimental.pallas import tpu_sc as plsc

NODES = 10000
PADN = 10240
EDGES = 320000
FEAT = 128
CHUNK = 128
NSC = 2
NSUB = 16
NW = NSC * NSUB
ROWS_PER_SUB = PADN // NSUB
ECHUNKS = 2560
EPAD = ECHUNKS * CHUNK
CPS = ECHUNKS // NW
NBUF = 4
ROUNDS = CPS // NBUF
CROWS = PADN // FEAT


@functools.cache
def _make_sc_agg(with_counts):
  mesh = plsc.VectorSubcoreMesh(
      core_axis_name="c", subcore_axis_name="s", num_cores=NSC,
      num_subcores=NSUB)

  out_type = [jax.ShapeDtypeStruct((NSC, PADN, FEAT), jnp.float32)]
  scratch = [
      pltpu.VMEM((CPS, CHUNK), jnp.int32),
      pltpu.VMEM((CPS, CHUNK), jnp.int32),
      pltpu.VMEM((CHUNK, FEAT), jnp.float32),
      pltpu.VMEM_SHARED((PADN, FEAT), jnp.float32),
      pltpu.SemaphoreType.DMA,
      pltpu.SemaphoreType.DMA,
      pltpu.SemaphoreType.DMA,
  ]
  if with_counts:
    out_type.append(jax.ShapeDtypeStruct((NSC, CROWS, FEAT), jnp.int32))
    scratch += [
        pltpu.VMEM((CROWS, FEAT), jnp.int32),
        pltpu.VMEM((CROWS,), jnp.int32),
        pltpu.VMEM_SHARED((CROWS, FEAT), jnp.int32),
    ]

  @functools.partial(
      pl.kernel, out_type=out_type, mesh=mesh, scratch_types=scratch,
      compiler_params=pltpu.CompilerParams(needs_layout_passes=False))
  def sc_agg(table, srci, dsti, *refs):
    if with_counts:
      (part, cntp, sidx, didx, rows, acc, isem0, isem1, gsem,
       cloc, iota_r, cacc) = refs
    else:
      part, sidx, didx, rows, acc, isem0, isem1, gsem = refs
    c = lax.axis_index("c")
    s = lax.axis_index("s")
    w = c * NSUB + s

    pltpu.async_copy(srci.at[pl.ds(w * CPS, CPS)], sidx, isem0)
    pltpu.async_copy(dsti.at[pl.ds(w * CPS, CPS)], didx, isem1)

    zv = jnp.zeros((16,), jnp.float32)

    def zrow(i, carry):
      def zcol(j, carry2):
        rows[i, pl.ds(j * 16, 16)] = zv
        return carry2
      return lax.fori_loop(0, FEAT // 16, zcol, carry)

    lax.fori_loop(0, 128, zrow, 0)
    for r in range(ROWS_PER_SUB // 128):
      pltpu.sync_copy(rows, acc.at[pl.ds(s * ROWS_PER_SUB + r * 128, 128)])

    if with_counts:
      zi = jnp.zeros((16,), jnp.int32)

      def czrow(i, carry):
        def czcol(j, carry2):
          cloc[i, pl.ds(j * 16, 16)] = zi
          return carry2
        return lax.fori_loop(0, FEAT // 16, czcol, carry)

      lax.fori_loop(0, CROWS, czrow, 0)
      for k in range(CROWS // 16):
        iota_r[pl.ds(k * 16, 16)] = lax.iota(jnp.int32, 16) + k * 16
      @pl.when(s < CROWS // 8)
      def _():
        pltpu.sync_copy(cloc.at[pl.ds(0, 8)], cacc.at[pl.ds(s * 8, 8)])
    plsc.subcore_barrier()

    pltpu.make_async_copy(srci.at[pl.ds(w * CPS, CPS)], sidx, isem0).wait()
    pltpu.make_async_copy(dsti.at[pl.ds(w * CPS, CPS)], didx, isem1).wait()

    def counts_for(k):
      for i in range(CHUNK // 16):
        v = didx[k, pl.ds(i * 16, 16)]
        row = lax.shift_right_logical(v, 7)
        col = jnp.bitwise_and(v, 127)
        cv, last = plsc.scan_count(v)
        plsc.addupdate_scatter(cloc, [row, col], cv, mask=last)

    def chunk_body(k, carry):
      cpy = pltpu.async_copy(table.at[sidx.at[k]], rows, gsem)
      if with_counts:
        counts_for(k)
      cpy.wait()
      pltpu.sync_copy(rows, acc.at[didx.at[k]], add=True)
      return carry

    lax.fori_loop(0, CPS, chunk_body, 0)

    if with_counts:
      pltpu.sync_copy(cloc, cacc.at[iota_r], add=True)
    plsc.subcore_barrier()

    pltpu.sync_copy(
        acc.at[pl.ds(s * ROWS_PER_SUB, ROWS_PER_SUB)],
        part.at[c, pl.ds(s * ROWS_PER_SUB, ROWS_PER_SUB)])
    if with_counts:
      @pl.when(s < CROWS // 8)
      def _():
        pltpu.sync_copy(cacc.at[pl.ds(s * 8, 8)],
                        cntp.at[c, pl.ds(s * 8, 8)])

  return sc_agg


_BLK = 1024
_GRID = (-(-NODES // _BLK),)


def _tc1_body(x_ref, wl_ref, wr_ref, b_ref, xl_ref, xr_ref):
  xb = x_ref[...]
  xl_ref[...] = jnp.dot(xb, wl_ref[...], preferred_element_type=jnp.float32)
  xr_ref[...] = (
      jnp.dot(xb, wr_ref[...], preferred_element_type=jnp.float32)
      + b_ref[...])


_tc1 = pl.pallas_call(
    _tc1_body,
    grid=_GRID,
    in_specs=[
        pl.BlockSpec((_BLK, FEAT), lambda i: (i, 0)),
        pl.BlockSpec((FEAT, FEAT), lambda i: (0, 0)),
        pl.BlockSpec((FEAT, FEAT), lambda i: (0, 0)),
        pl.BlockSpec((1, FEAT), lambda i: (0, 0)),
    ],
    out_specs=[
        pl.BlockSpec((_BLK, FEAT), lambda i: (i, 0)),
        pl.BlockSpec((_BLK, FEAT), lambda i: (i, 0)),
    ],
    out_shape=[
        jax.ShapeDtypeStruct((NODES, FEAT), jnp.float32),
        jax.ShapeDtypeStruct((NODES, FEAT), jnp.float32),
    ],
)


def _tc2_body(p0_ref, p1_ref, cnt_ref, xr_ref, wl_ref, wr_ref, b_ref,
              xl2_ref, xr2_ref):
  inv = 1.0 / jnp.maximum(cnt_ref[...], 1.0)
  x1 = (p0_ref[...] + p1_ref[...]) * inv + xr_ref[...]
  xl2_ref[...] = jnp.dot(x1, wl_ref[...], preferred_element_type=jnp.float32)
  xr2_ref[...] = (
      jnp.dot(x1, wr_ref[...], preferred_element_type=jnp.float32)
      + b_ref[...])


_tc2 = pl.pallas_call(
    _tc2_body,
    grid=_GRID,
    in_specs=[
        pl.BlockSpec((_BLK, FEAT), lambda i: (i, 0)),
        pl.BlockSpec((_BLK, FEAT), lambda i: (i, 0)),
        pl.BlockSpec((_BLK, FEAT), lambda i: (i, 0)),
        pl.BlockSpec((_BLK, FEAT), lambda i: (i, 0)),
        pl.BlockSpec((FEAT, FEAT), lambda i: (0, 0)),
        pl.BlockSpec((FEAT, FEAT), lambda i: (0, 0)),
        pl.BlockSpec((1, FEAT), lambda i: (0, 0)),
    ],
    out_specs=[
        pl.BlockSpec((_BLK, FEAT), lambda i: (i, 0)),
        pl.BlockSpec((_BLK, FEAT), lambda i: (i, 0)),
    ],
    out_shape=[
        jax.ShapeDtypeStruct((NODES, FEAT), jnp.float32),
        jax.ShapeDtypeStruct((NODES, FEAT), jnp.float32),
    ],
)


def _tc3_body(q0_ref, q1_ref, cnt_ref, xr_ref, out_ref):
  inv = 1.0 / jnp.maximum(cnt_ref[...], 1.0)
  out_ref[...] = (q0_ref[...] + q1_ref[...]) * inv + xr_ref[...]


_tc3 = pl.pallas_call(
    _tc3_body,
    grid=_GRID,
    in_specs=[
        pl.BlockSpec((_BLK, FEAT), lambda i: (i, 0)),
        pl.BlockSpec((_BLK, FEAT), lambda i: (i, 0)),
        pl.BlockSpec((_BLK, FEAT), lambda i: (i, 0)),
        pl.BlockSpec((_BLK, FEAT), lambda i: (i, 0)),
    ],
    out_specs=pl.BlockSpec((_BLK, FEAT), lambda i: (i, 0)),
    out_shape=jax.ShapeDtypeStruct((NODES, FEAT), jnp.float32),
)


def kernel(x, edge_index, W1_l, b1_l, W1_r, W2_l, b2_l, W2_r):
  src = edge_index[0]
  dst = edge_index[1]

  pe = EPAD - EDGES
  pad_i = jnp.arange(pe, dtype=jnp.int32)
  src2d = jnp.concatenate([src, pad_i % NODES]).reshape(ECHUNKS, CHUNK)
  dst2d = jnp.concatenate(
      [dst, NODES + pad_i % (PADN - NODES)]).reshape(ECHUNKS, CHUNK)

  xl1, xr1b = _tc1(x, W1_l, W1_r, b1_l.reshape(1, FEAT))

  part1, cntp = _make_sc_agg(True)(xl1, src2d, dst2d)

  p0 = part1[0, :NODES]
  p1 = part1[1, :NODES]
  cnt = (cntp[0] + cntp[1]).astype(jnp.float32).reshape(PADN)[:NODES]
  cnt_b = jnp.broadcast_to(cnt[:, None], (NODES, FEAT))

  xl2, xr2b = _tc2(p0, p1, cnt_b, xr1b, W2_l, W2_r, b2_l.reshape(1, FEAT))

  (part2,) = _make_sc_agg(False)(xl2, src2d, dst2d)

  return _tc3(part2[0, :NODES], part2[1, :NODES], cnt_b, xr2b)


# speedup vs baseline: 9.1419x; 1.0293x over previous
import functools

import jax
import jax.numpy as jnp
from jax import lax
from jax.experimental import pallas as pl
from jax.experimental.pallas import tpu as pltpu
from jax.experimental.pallas import tpu_sc as plsc

NODES = 10000
PADN = 10240
EDGES = 320000
FEAT = 128
CHUNK = 128
NSC = 2
NSUB = 16
NW = NSC * NSUB
ROWS_PER_SUB = PADN // NSUB
ECHUNKS = 2560
EPAD = ECHUNKS * CHUNK
CPS = ECHUNKS // NW
NBUF = 4
ROUNDS = CPS // NBUF
CROWS = PADN // FEAT


@functools.cache
def _make_sc_agg(with_counts):
  mesh = plsc.VectorSubcoreMesh(
      core_axis_name="c", subcore_axis_name="s", num_cores=NSC,
      num_subcores=NSUB)

  out_type = [jax.ShapeDtypeStruct((NSC, PADN, FEAT), jnp.float32)]
  scratch = [
      pltpu.VMEM((CPS, CHUNK), jnp.int32),
      pltpu.VMEM((CPS, CHUNK), jnp.int32),
      pltpu.VMEM((CHUNK, FEAT), jnp.float32),
      pltpu.VMEM_SHARED((PADN, FEAT), jnp.float32),
      pltpu.SemaphoreType.DMA,
      pltpu.SemaphoreType.DMA,
      pltpu.SemaphoreType.DMA,
  ]
  if with_counts:
    out_type.append(jax.ShapeDtypeStruct((NSC, CROWS, FEAT), jnp.int32))
    scratch += [
        pltpu.VMEM((CROWS, FEAT), jnp.int32),
        pltpu.VMEM((CROWS,), jnp.int32),
        pltpu.VMEM_SHARED((CROWS, FEAT), jnp.int32),
    ]

  @functools.partial(
      pl.kernel, out_type=out_type, mesh=mesh, scratch_types=scratch,
      compiler_params=pltpu.CompilerParams(needs_layout_passes=False))
  def sc_agg(table, srci, dsti, *refs):
    if with_counts:
      (part, cntp, sidx, didx, rows, acc, isem0, isem1, gsem,
       cloc, iota_r, cacc) = refs
    else:
      part, sidx, didx, rows, acc, isem0, isem1, gsem = refs
    c = lax.axis_index("c")
    s = lax.axis_index("s")
    w = c * NSUB + s

    pltpu.async_copy(srci.at[pl.ds(w * CPS, CPS)], sidx, isem0)
    pltpu.async_copy(dsti.at[pl.ds(w * CPS, CPS)], didx, isem1)

    zv = jnp.zeros((16,), jnp.float32)

    def zrow(i, carry):
      def zcol(j, carry2):
        rows[i, pl.ds(j * 16, 16)] = zv
        return carry2
      return lax.fori_loop(0, FEAT // 16, zcol, carry)

    lax.fori_loop(0, 128, zrow, 0)
    for r in range(ROWS_PER_SUB // 128):
      pltpu.sync_copy(rows, acc.at[pl.ds(s * ROWS_PER_SUB + r * 128, 128)])

    if with_counts:
      zi = jnp.zeros((16,), jnp.int32)

      def czrow(i, carry):
        def czcol(j, carry2):
          cloc[i, pl.ds(j * 16, 16)] = zi
          return carry2
        return lax.fori_loop(0, FEAT // 16, czcol, carry)

      lax.fori_loop(0, CROWS, czrow, 0)
      for k in range(CROWS // 16):
        iota_r[pl.ds(k * 16, 16)] = lax.iota(jnp.int32, 16) + k * 16
      @pl.when(s < CROWS // 8)
      def _():
        pltpu.sync_copy(cloc.at[pl.ds(0, 8)], cacc.at[pl.ds(s * 8, 8)])
    plsc.subcore_barrier()

    pltpu.make_async_copy(srci.at[pl.ds(w * CPS, CPS)], sidx, isem0).wait()
    pltpu.make_async_copy(dsti.at[pl.ds(w * CPS, CPS)], didx, isem1).wait()

    def counts_for(k):
      for i in range(CHUNK // 16):
        v = didx[k, pl.ds(i * 16, 16)]
        row = lax.shift_right_logical(v, 7)
        col = jnp.bitwise_and(v, 127)
        cv, last = plsc.scan_count(v)
        plsc.addupdate_scatter(cloc, [row, col], cv, mask=last)

    def chunk_body(k, carry):
      cpy = pltpu.async_copy(table.at[sidx.at[k]], rows, gsem)
      if with_counts:
        counts_for(k)
      cpy.wait()
      pltpu.sync_copy(rows, acc.at[didx.at[k]], add=True)
      return carry

    lax.fori_loop(0, CPS, chunk_body, 0)

    if with_counts:
      pltpu.sync_copy(cloc, cacc.at[iota_r], add=True)
    plsc.subcore_barrier()

    pltpu.sync_copy(
        acc.at[pl.ds(s * ROWS_PER_SUB, ROWS_PER_SUB)],
        part.at[c, pl.ds(s * ROWS_PER_SUB, ROWS_PER_SUB)])
    if with_counts:
      @pl.when(s < CROWS // 8)
      def _():
        pltpu.sync_copy(cacc.at[pl.ds(s * 8, 8)],
                        cntp.at[c, pl.ds(s * 8, 8)])

  return sc_agg


_BLK = 1024
_GRID = (-(-NODES // _BLK),)


def _tc1_body(x_ref, wl_ref, wr_ref, b_ref, xl_ref, xr_ref):
  xb = x_ref[...]
  xl_ref[...] = jnp.dot(xb, wl_ref[...], preferred_element_type=jnp.float32)
  xr_ref[...] = (
      jnp.dot(xb, wr_ref[...], preferred_element_type=jnp.float32)
      + b_ref[...])


_tc1 = pl.pallas_call(
    _tc1_body,
    grid=_GRID,
    in_specs=[
        pl.BlockSpec((_BLK, FEAT), lambda i: (i, 0)),
        pl.BlockSpec((FEAT, FEAT), lambda i: (0, 0)),
        pl.BlockSpec((FEAT, FEAT), lambda i: (0, 0)),
        pl.BlockSpec((1, FEAT), lambda i: (0, 0)),
    ],
    out_specs=[
        pl.BlockSpec((_BLK, FEAT), lambda i: (i, 0)),
        pl.BlockSpec((_BLK, FEAT), lambda i: (i, 0)),
    ],
    out_shape=[
        jax.ShapeDtypeStruct((NODES, FEAT), jnp.float32),
        jax.ShapeDtypeStruct((NODES, FEAT), jnp.float32),
    ],
)


def _tc2_body(p0_ref, p1_ref, cnt_ref, xr_ref, wl_ref, wr_ref, b_ref,
              xl2_ref, xr2_ref):
  inv = 1.0 / jnp.maximum(cnt_ref[...], 1.0)
  x1 = (p0_ref[0] + p1_ref[0]) * inv + xr_ref[...]
  xl2_ref[...] = jnp.dot(x1, wl_ref[...], preferred_element_type=jnp.float32)
  xr2_ref[...] = (
      jnp.dot(x1, wr_ref[...], preferred_element_type=jnp.float32)
      + b_ref[...])


_tc2 = pl.pallas_call(
    _tc2_body,
    grid=_GRID,
    in_specs=[
        pl.BlockSpec((1, _BLK, FEAT), lambda i: (0, i, 0)),
        pl.BlockSpec((1, _BLK, FEAT), lambda i: (1, i, 0)),
        pl.BlockSpec((_BLK, FEAT), lambda i: (i, 0)),
        pl.BlockSpec((_BLK, FEAT), lambda i: (i, 0)),
        pl.BlockSpec((FEAT, FEAT), lambda i: (0, 0)),
        pl.BlockSpec((FEAT, FEAT), lambda i: (0, 0)),
        pl.BlockSpec((1, FEAT), lambda i: (0, 0)),
    ],
    out_specs=[
        pl.BlockSpec((_BLK, FEAT), lambda i: (i, 0)),
        pl.BlockSpec((_BLK, FEAT), lambda i: (i, 0)),
    ],
    out_shape=[
        jax.ShapeDtypeStruct((NODES, FEAT), jnp.float32),
        jax.ShapeDtypeStruct((NODES, FEAT), jnp.float32),
    ],
)


def _tc3_body(q0_ref, q1_ref, cnt_ref, xr_ref, out_ref):
  inv = 1.0 / jnp.maximum(cnt_ref[...], 1.0)
  out_ref[...] = (q0_ref[0] + q1_ref[0]) * inv + xr_ref[...]


_tc3 = pl.pallas_call(
    _tc3_body,
    grid=_GRID,
    in_specs=[
        pl.BlockSpec((1, _BLK, FEAT), lambda i: (0, i, 0)),
        pl.BlockSpec((1, _BLK, FEAT), lambda i: (1, i, 0)),
        pl.BlockSpec((_BLK, FEAT), lambda i: (i, 0)),
        pl.BlockSpec((_BLK, FEAT), lambda i: (i, 0)),
    ],
    out_specs=pl.BlockSpec((_BLK, FEAT), lambda i: (i, 0)),
    out_shape=jax.ShapeDtypeStruct((NODES, FEAT), jnp.float32),
)


def kernel(x, edge_index, W1_l, b1_l, W1_r, W2_l, b2_l, W2_r):
  src = edge_index[0]
  dst = edge_index[1]

  pe = EPAD - EDGES
  pad_i = jnp.arange(pe, dtype=jnp.int32)
  src2d = jnp.concatenate([src, pad_i % NODES]).reshape(ECHUNKS, CHUNK)
  dst2d = jnp.concatenate(
      [dst, NODES + pad_i % (PADN - NODES)]).reshape(ECHUNKS, CHUNK)

  xl1, xr1b = _tc1(x, W1_l, W1_r, b1_l.reshape(1, FEAT))

  part1, cntp = _make_sc_agg(True)(xl1, src2d, dst2d)

  cnt = (cntp[0] + cntp[1]).astype(jnp.float32).reshape(PADN)[:NODES]
  cnt_b = jnp.broadcast_to(cnt[:, None], (NODES, FEAT))

  xl2, xr2b = _tc2(part1, part1, cnt_b, xr1b, W2_l, W2_r,
                   b2_l.reshape(1, FEAT))

  (part2,) = _make_sc_agg(False)(xl2, src2d, dst2d)

  return _tc3(part2, part2, cnt_b, xr2b)


# speedup vs baseline: 9.1469x; 1.0006x over previous
import functools

import jax
import jax.numpy as jnp
from jax import lax
from jax.experimental import pallas as pl
from jax.experimental.pallas import tpu as pltpu
from jax.experimental.pallas import tpu_sc as plsc

NODES = 10000
PADN = 10240
EDGES = 320000
FEAT = 128
CHUNK = 128
NSC = 2
NSUB = 16
NW = NSC * NSUB
ROWS_PER_SUB = PADN // NSUB
ECHUNKS = 2560
EPAD = ECHUNKS * CHUNK
CPS = ECHUNKS // NW
CROWS = PADN // FEAT


@functools.cache
def _make_sc_agg(with_counts):
  mesh = plsc.VectorSubcoreMesh(
      core_axis_name="c", subcore_axis_name="s", num_cores=NSC,
      num_subcores=NSUB)

  out_type = [jax.ShapeDtypeStruct((NSC, PADN, FEAT), jnp.float32)]
  scratch = [
      pltpu.VMEM((CPS, CHUNK), jnp.int32),
      pltpu.VMEM((CPS, CHUNK), jnp.int32),
      pltpu.VMEM((CHUNK, FEAT), jnp.float32),
      pltpu.VMEM_SHARED((PADN, FEAT), jnp.float32),
      pltpu.SemaphoreType.DMA,
      pltpu.SemaphoreType.DMA,
      pltpu.SemaphoreType.DMA,
  ]
  if with_counts:
    out_type.append(jax.ShapeDtypeStruct((NSC, CROWS, FEAT), jnp.int32))
    scratch += [
        pltpu.VMEM((CROWS, FEAT), jnp.int32),
        pltpu.VMEM((CROWS,), jnp.int32),
        pltpu.VMEM_SHARED((CROWS, FEAT), jnp.int32),
    ]

  @functools.partial(
      pl.kernel, out_type=out_type, mesh=mesh, scratch_types=scratch,
      compiler_params=pltpu.CompilerParams(needs_layout_passes=False))
  def sc_agg(table, srci, dsti, *refs):
    if with_counts:
      (part, cntp, sidx, didx, rows, acc, isem0, isem1, gsem,
       cloc, iota_r, cacc) = refs
    else:
      part, sidx, didx, rows, acc, isem0, isem1, gsem = refs
    c = lax.axis_index("c")
    s = lax.axis_index("s")
    w = c * NSUB + s

    pltpu.async_copy(srci.at[pl.ds(w * CPS, CPS)], sidx, isem0)
    pltpu.async_copy(dsti.at[pl.ds(w * CPS, CPS)], didx, isem1)

    zv = jnp.zeros((16,), jnp.float32)

    def zrow(i, carry):
      def zcol(j, carry2):
        rows[i, pl.ds(j * 16, 16)] = zv
        return carry2
      return lax.fori_loop(0, FEAT // 16, zcol, carry)

    lax.fori_loop(0, 128, zrow, 0)
    for r in range(ROWS_PER_SUB // 128):
      pltpu.sync_copy(rows, acc.at[pl.ds(s * ROWS_PER_SUB + r * 128, 128)])

    if with_counts:
      zi = jnp.zeros((16,), jnp.int32)

      def czrow(i, carry):
        def czcol(j, carry2):
          cloc[i, pl.ds(j * 16, 16)] = zi
          return carry2
        return lax.fori_loop(0, FEAT // 16, czcol, carry)

      lax.fori_loop(0, CROWS, czrow, 0)
      for k in range(CROWS // 16):
        iota_r[pl.ds(k * 16, 16)] = lax.iota(jnp.int32, 16) + k * 16
      @pl.when(s < CROWS // 8)
      def _():
        pltpu.sync_copy(cloc.at[pl.ds(0, 8)], cacc.at[pl.ds(s * 8, 8)])
    plsc.subcore_barrier()

    pltpu.make_async_copy(srci.at[pl.ds(w * CPS, CPS)], sidx, isem0).wait()
    pltpu.make_async_copy(dsti.at[pl.ds(w * CPS, CPS)], didx, isem1).wait()

    def counts_for(k):
      for i in range(CHUNK // 16):
        v = didx[k, pl.ds(i * 16, 16)]
        row = lax.shift_right_logical(v, 7)
        col = jnp.bitwise_and(v, 127)
        cv, last = plsc.scan_count(v)
        plsc.addupdate_scatter(cloc, [row, col], cv, mask=last)

    def chunk_body(k, carry):
      cpy = pltpu.async_copy(table.at[sidx.at[k]], rows, gsem)
      if with_counts:
        counts_for(k)
      cpy.wait()
      pltpu.sync_copy(rows, acc.at[didx.at[k]], add=True)
      return carry

    lax.fori_loop(0, CPS, chunk_body, 0)

    if with_counts:
      pltpu.sync_copy(cloc, cacc.at[iota_r], add=True)
    plsc.subcore_barrier()

    pltpu.sync_copy(
        acc.at[pl.ds(s * ROWS_PER_SUB, ROWS_PER_SUB)],
        part.at[c, pl.ds(s * ROWS_PER_SUB, ROWS_PER_SUB)])
    if with_counts:
      @pl.when(s < CROWS // 8)
      def _():
        pltpu.sync_copy(cacc.at[pl.ds(s * 8, 8)],
                        cntp.at[c, pl.ds(s * 8, 8)])

  return sc_agg


_BLK = 1024
_GRID = (-(-NODES // _BLK),)


def _tc1_body(x_ref, wl_ref, wr_ref, b_ref, xl_ref, xr_ref):
  xb = x_ref[...]
  xl_ref[...] = jnp.dot(xb, wl_ref[...], preferred_element_type=jnp.float32)
  xr_ref[...] = (
      jnp.dot(xb, wr_ref[...], preferred_element_type=jnp.float32)
      + b_ref[...])


_tc1 = pl.pallas_call(
    _tc1_body,
    grid=_GRID,
    in_specs=[
        pl.BlockSpec((_BLK, FEAT), lambda i: (i, 0)),
        pl.BlockSpec((FEAT, FEAT), lambda i: (0, 0)),
        pl.BlockSpec((FEAT, FEAT), lambda i: (0, 0)),
        pl.BlockSpec((1, FEAT), lambda i: (0, 0)),
    ],
    out_specs=[
        pl.BlockSpec((_BLK, FEAT), lambda i: (i, 0)),
        pl.BlockSpec((_BLK, FEAT), lambda i: (i, 0)),
    ],
    out_shape=[
        jax.ShapeDtypeStruct((NODES, FEAT), jnp.float32),
        jax.ShapeDtypeStruct((NODES, FEAT), jnp.float32),
    ],
)


def _tc2_body(p0_ref, p1_ref, cnt_ref, xr_ref, wl_ref, wr_ref, b_ref,
              xl2_ref, xr2_ref):
  inv = 1.0 / jnp.maximum(cnt_ref[...], 1.0)
  x1 = (p0_ref[0] + p1_ref[0]) * inv + xr_ref[...]
  xl2_ref[...] = jnp.dot(x1, wl_ref[...], preferred_element_type=jnp.float32)
  xr2_ref[...] = (
      jnp.dot(x1, wr_ref[...], preferred_element_type=jnp.float32)
      + b_ref[...])


_tc2 = pl.pallas_call(
    _tc2_body,
    grid=_GRID,
    in_specs=[
        pl.BlockSpec((1, _BLK, FEAT), lambda i: (0, i, 0)),
        pl.BlockSpec((1, _BLK, FEAT), lambda i: (1, i, 0)),
        pl.BlockSpec((_BLK, FEAT), lambda i: (i, 0)),
        pl.BlockSpec((_BLK, FEAT), lambda i: (i, 0)),
        pl.BlockSpec((FEAT, FEAT), lambda i: (0, 0)),
        pl.BlockSpec((FEAT, FEAT), lambda i: (0, 0)),
        pl.BlockSpec((1, FEAT), lambda i: (0, 0)),
    ],
    out_specs=[
        pl.BlockSpec((_BLK, FEAT), lambda i: (i, 0)),
        pl.BlockSpec((_BLK, FEAT), lambda i: (i, 0)),
    ],
    out_shape=[
        jax.ShapeDtypeStruct((NODES, FEAT), jnp.float32),
        jax.ShapeDtypeStruct((NODES, FEAT), jnp.float32),
    ],
)


def _tc3_body(q0_ref, q1_ref, cnt_ref, xr_ref, out_ref):
  inv = 1.0 / jnp.maximum(cnt_ref[...], 1.0)
  out_ref[...] = (q0_ref[0] + q1_ref[0]) * inv + xr_ref[...]


_tc3 = pl.pallas_call(
    _tc3_body,
    grid=_GRID,
    in_specs=[
        pl.BlockSpec((1, _BLK, FEAT), lambda i: (0, i, 0)),
        pl.BlockSpec((1, _BLK, FEAT), lambda i: (1, i, 0)),
        pl.BlockSpec((_BLK, FEAT), lambda i: (i, 0)),
        pl.BlockSpec((_BLK, FEAT), lambda i: (i, 0)),
    ],
    out_specs=pl.BlockSpec((_BLK, FEAT), lambda i: (i, 0)),
    out_shape=jax.ShapeDtypeStruct((NODES, FEAT), jnp.float32),
)


def kernel(x, edge_index, W1_l, b1_l, W1_r, W2_l, b2_l, W2_r):
  src = edge_index[0]
  dst = edge_index[1]

  pe = EPAD - EDGES
  pad_i = jnp.arange(pe, dtype=jnp.int32)
  src2d = jnp.concatenate([src, pad_i % NODES]).reshape(ECHUNKS, CHUNK)
  dst2d = jnp.concatenate(
      [dst, NODES + pad_i % (PADN - NODES)]).reshape(ECHUNKS, CHUNK)

  xl1, xr1b = _tc1(x, W1_l, W1_r, b1_l.reshape(1, FEAT))

  part1, cntp = _make_sc_agg(True)(xl1, src2d, dst2d)

  cnt = (cntp[0] + cntp[1]).astype(jnp.float32).reshape(PADN)[:NODES]
  cnt_b = jnp.broadcast_to(cnt[:, None], (NODES, FEAT))

  xl2, xr2b = _tc2(part1, part1, cnt_b, xr1b, W2_l, W2_r,
                   b2_l.reshape(1, FEAT))

  (part2,) = _make_sc_agg(False)(xl2, src2d, dst2d)

  return _tc3(part2, part2, cnt_b, xr2b)
